# Initial kernel scaffold; baseline (speedup 1.0000x reference)
import jax
import jax.numpy as jnp
from jax.experimental import pallas as pl


def kernel(freq_positions, time_positions, freq_relative_emb, time_relative_emb, W_dist, b_dist):
    raise NotImplementedError("write your pallas kernel here")


# speedup vs baseline: 4.9322x; 4.9322x over previous
import jax
import jax.numpy as jnp
from jax.experimental import pallas as pl

B, S = 256, 512
EMBED = 768
D4 = EMBED // 4
D2 = EMBED // 2
NF, NT = 8, 64
R = 8


def _body(fp_ref, tp_ref, ft_ref, tt_ref, w_ref, b_ref, out_ref):
    fp = fp_ref[0]
    tp = tp_ref[0]
    fc = jnp.mean(fp, axis=0, keepdims=True)
    tc = jnp.mean(tp, axis=0, keepdims=True)
    rf = fp - fc
    rt = tp - tc
    dist = rf * w_ref[0:1, :] + rt * w_ref[1:2, :] + b_ref[...]
    fi = jnp.clip(rf + (NF - 1), 0, 2 * NF - 2).astype(jnp.int32)
    ti = jnp.clip(rt + (NT - 1), 0, 2 * NT - 2).astype(jnp.int32)
    ohf = (fi == jax.lax.broadcasted_iota(jnp.int32, (1, 16), 1)).astype(jnp.float32)
    oht = (ti == jax.lax.broadcasted_iota(jnp.int32, (1, 128), 1)).astype(jnp.float32)
    fe = jnp.dot(ohf, ft_ref[...], preferred_element_type=jnp.float32)
    te = jnp.dot(oht, tt_ref[...], preferred_element_type=jnp.float32)
    out = jnp.concatenate([dist, fe, te], axis=-1)
    out_ref[...] = out.reshape(1, S, EMBED)


def kernel(freq_positions, time_positions, freq_relative_emb, time_relative_emb, W_dist, b_dist):
    fp = freq_positions.astype(jnp.float32)[..., None]
    tp = time_positions.astype(jnp.float32)[..., None]
    ft = jnp.pad(freq_relative_emb, ((0, 1), (0, 0)))
    tt = jnp.pad(time_relative_emb, ((0, 1), (0, 0)))
    b2 = b_dist.reshape(1, D2)
    return pl.pallas_call(
        _body,
        grid=(B,),
        in_specs=[
            pl.BlockSpec((1, S, 1), lambda i: (i, 0, 0)),
            pl.BlockSpec((1, S, 1), lambda i: (i, 0, 0)),
            pl.BlockSpec((16, D4), lambda i: (0, 0)),
            pl.BlockSpec((128, D4), lambda i: (0, 0)),
            pl.BlockSpec((2, D2), lambda i: (0, 0)),
            pl.BlockSpec((1, D2), lambda i: (0, 0)),
        ],
        out_specs=pl.BlockSpec((1, S, EMBED), lambda i: (i, 0, 0)),
        out_shape=jax.ShapeDtypeStruct((B, S, EMBED), jnp.float32),
    )(fp, tp, ft, tt, W_dist, b2)


# speedup vs baseline: 5.0876x; 1.0315x over previous
import jax
import jax.numpy as jnp
from jax.experimental import pallas as pl

B, S = 256, 512
EMBED = 768
D4 = EMBED // 4
D2 = EMBED // 2
NF, NT = 8, 64
R = 8


def _body(fp_ref, tp_ref, g_ref, w_ref, b_ref, out_ref):
    fp = fp_ref[0]
    tp = tp_ref[0]
    fc = jnp.mean(fp, axis=0, keepdims=True)
    tc = jnp.mean(tp, axis=0, keepdims=True)
    rf = fp - fc
    rt = tp - tc
    fi = jnp.clip(rf + (NF - 1), 0, 2 * NF - 2).astype(jnp.int32)
    ti = jnp.clip(rt + (NT - 1), 0, 2 * NT - 2).astype(jnp.int32)
    df = (rf + (NF - 1))[0:1, 0:1] - fi[0:1, 0:1].astype(jnp.float32)
    dt = (rt + (NT - 1))[0:1, 0:1] - ti[0:1, 0:1].astype(jnp.float32)
    oht = (ti == jax.lax.broadcasted_iota(jnp.int32, (1, 128), 1)).astype(jnp.float32)
    ohf = (fi == jax.lax.broadcasted_iota(jnp.int32, (1, 16), 1)).astype(jnp.float32)
    x = jnp.concatenate([oht, ohf], axis=1)
    r = df * w_ref[0:1, :] + dt * w_ref[1:2, :] + b_ref[...]
    rpad = jnp.concatenate([r, jnp.zeros((1, D2), jnp.float32)], axis=1)
    out = jnp.dot(x, g_ref[...], preferred_element_type=jnp.float32) + rpad
    out_ref[...] = out.reshape(1, S, EMBED)


def kernel(freq_positions, time_positions, freq_relative_emb, time_relative_emb, W_dist, b_dist):
    fp = freq_positions.astype(jnp.float32)[..., None]
    tp = time_positions.astype(jnp.float32)[..., None]
    vf = (jnp.arange(16, dtype=jnp.float32) - (NF - 1))[:, None]
    vt = (jnp.arange(128, dtype=jnp.float32) - (NT - 1))[:, None]
    z = jnp.zeros((16, D4), jnp.float32)
    zt = jnp.zeros((128, D4), jnp.float32)
    ftp = jnp.pad(freq_relative_emb, ((0, 1), (0, 0)))
    ttp = jnp.pad(time_relative_emb, ((0, 1), (0, 0)))
    g1 = jnp.concatenate([vf * W_dist[0:1, :], ftp, z], axis=1)
    g2 = jnp.concatenate([vt * W_dist[1:2, :], zt, ttp], axis=1)
    g = jnp.concatenate([g2, g1], axis=0)
    b2 = b_dist.reshape(1, D2)
    return pl.pallas_call(
        _body,
        grid=(B,),
        in_specs=[
            pl.BlockSpec((1, S, 1), lambda i: (i, 0, 0)),
            pl.BlockSpec((1, S, 1), lambda i: (i, 0, 0)),
            pl.BlockSpec((144, EMBED), lambda i: (0, 0)),
            pl.BlockSpec((2, D2), lambda i: (0, 0)),
            pl.BlockSpec((1, D2), lambda i: (0, 0)),
        ],
        out_specs=pl.BlockSpec((1, S, EMBED), lambda i: (i, 0, 0)),
        out_shape=jax.ShapeDtypeStruct((B, S, EMBED), jnp.float32),
    )(fp, tp, g, W_dist, b2)
